# Initial kernel scaffold; baseline (speedup 1.0000x reference)
import jax
import jax.numpy as jnp
from jax.experimental import pallas as pl


def kernel(sparse_inputs, dense_inputs, tables, W):
    raise NotImplementedError("write your pallas kernel here")


# speedup vs baseline: 1.0541x; 1.0541x over previous
import functools

import jax
import jax.numpy as jnp
from jax import lax
from jax.experimental import pallas as pl
from jax.experimental.pallas import tpu as pltpu
from jax.experimental.pallas import tpu_sc as plsc

B, F, V, D, DD = 16384, 26, 100000, 32, 13
NC, NS, L = 2, 16, 16
NW = NC * NS
BPW = B // NW
IPW = BPW * F
STEP_ROWS = 4 * F
NSTEPS = IPW // STEP_ROWS


def _sc_gather(tables_flat, sparse_flat):
    mesh = plsc.VectorSubcoreMesh(core_axis_name="c", subcore_axis_name="s")

    @functools.partial(
        pl.kernel,
        mesh=mesh,
        out_type=jax.ShapeDtypeStruct((B * F, D), jnp.float32),
        scratch_types=[
            pltpu.VMEM((IPW,), jnp.int32),
            pltpu.VMEM((STEP_ROWS, D), jnp.float32),
            pltpu.SemaphoreType.DMA,
        ],
        compiler_params=pltpu.CompilerParams(use_tc_tiling_on_sc=False),
    )
    def k(tbl_hbm, idx_hbm, out_hbm, idx_v, rows_v, sem):
        wid = lax.axis_index("s") * NC + lax.axis_index("c")
        ibase = wid * IPW
        pltpu.sync_copy(idx_hbm.at[pl.ds(ibase, IPW)], idx_v)

        def conv(i, _):
            pos = i * L + lax.iota(jnp.int32, L)
            off = lax.rem(pos, F) * V
            idx_v[pl.ds(i * L, L)] = idx_v[pl.ds(i * L, L)] + off
            return _
        lax.fori_loop(0, IPW // L, conv, None)

        def step(s, _):
            pltpu.async_copy(
                tbl_hbm.at[idx_v.at[pl.ds(s * STEP_ROWS, STEP_ROWS)]],
                rows_v, sem).wait()
            pltpu.sync_copy(rows_v,
                            out_hbm.at[pl.ds(ibase + s * STEP_ROWS, STEP_ROWS)])
            return _
        lax.fori_loop(0, NSTEPS, step, None)

    return k(tables_flat, sparse_flat)


def _tc_dense(dense_inputs, W):
    BB = 512

    def mm(x_ref, w_ref, o_ref):
        o_ref[...] = jnp.dot(x_ref[...], w_ref[...],
                             preferred_element_type=jnp.float32)

    return pl.pallas_call(
        mm,
        grid=(B // BB,),
        in_specs=[
            pl.BlockSpec((BB, DD), lambda i: (i, 0)),
            pl.BlockSpec((DD, DD * D), lambda i: (0, 0)),
        ],
        out_specs=pl.BlockSpec((BB, DD * D), lambda i: (i, 0)),
        out_shape=jax.ShapeDtypeStruct((B, DD * D), jnp.float32),
    )(dense_inputs, W)


def kernel(sparse_inputs, dense_inputs, tables, W):
    tables_flat = tables.reshape(F * V, D)
    sparse_flat = sparse_inputs.reshape(B * F).astype(jnp.int32)
    sparse_out = _sc_gather(tables_flat, sparse_flat).reshape(B, F, D)
    dense_out = _tc_dense(dense_inputs, W).reshape(B, DD, D)
    return jnp.concatenate([sparse_out, dense_out], axis=1)
